# baseline (speedup 1.0000x reference)
import functools

import jax
import jax.numpy as jnp
from jax import lax
from jax.experimental import pallas as pl
from jax.experimental.pallas import tpu as pltpu
from jax.experimental.pallas import tpu_sc as plsc

_N = 10000
_E = 320000
_D = 128
_R_CUT = 5.0


def _lin1_body(x_ref, wT_ref, o_ref):
    o_ref[...] = jnp.dot(x_ref[...], wT_ref[...],
                         preferred_element_type=jnp.float32)


def _lin1(x, wT):
    return pl.pallas_call(
        _lin1_body,
        out_shape=jax.ShapeDtypeStruct((_N, _D), jnp.float32),
    )(x, wT)


_BE = 512
_GE = _E // _BE


def _col128(row):
    rid = lax.broadcasted_iota(jnp.int32, (128, 128), 0)
    lid = lax.broadcasted_iota(jnp.int32, (128, 128), 1)
    b = jnp.broadcast_to(row, (128, 128))
    return jnp.sum(jnp.where(rid == lid, b, 0.0), axis=1, keepdims=True)


def _filter_body(attrs_ref, ew_ref, wf1T_ref, bf1_ref, wf2T_ref, w_ref):
    a = attrs_ref[...]
    h = jnp.dot(a, wf1T_ref[...], preferred_element_type=jnp.float32)
    h = h + bf1_ref[...]
    h = h * jax.nn.sigmoid(h)
    w = jnp.dot(h.astype(jnp.bfloat16), wf2T_ref[...],
                preferred_element_type=jnp.float32)
    ew = ew_ref[0]
    c = 0.5 * (jnp.cos(jnp.pi * ew / _R_CUT) + 1.0)
    c = c * (ew < _R_CUT).astype(jnp.float32)
    ccol = jnp.concatenate(
        [_col128(c[r:r + 1, :]) for r in range(_BE // 128)], axis=0)
    w_ref[...] = w * ccol


def _edge_filter(edge_attrs, edge_weights, wf1T, bf1, wf2T):
    ew2 = edge_weights.reshape(_GE, _BE // 128, 128)
    return pl.pallas_call(
        _filter_body,
        grid=(_GE,),
        in_specs=[
            pl.BlockSpec((_BE, _D), lambda i: (i, 0)),
            pl.BlockSpec((1, _BE // 128, 128), lambda i: (i, 0, 0)),
            pl.BlockSpec((_D, _D), lambda i: (0, 0)),
            pl.BlockSpec((1, _D), lambda i: (0, 0)),
            pl.BlockSpec((_D, _D), lambda i: (0, 0)),
        ],
        out_specs=pl.BlockSpec((_BE, _D), lambda i: (i, 0)),
        out_shape=jax.ShapeDtypeStruct((_E, _D), jnp.float32),
    )(edge_attrs, ew2, wf1T, bf1.reshape(1, _D), wf2T)


_NC, _NS, _L = 2, 16, 16
_NW = _NC * _NS
_CH = 80
_PERW = _E // _NW
_NCH = _PERW // _CH
_NP = 10240
_RPT = _NP // _NS

_sc_mesh = plsc.VectorSubcoreMesh(core_axis_name="c", subcore_axis_name="s")


@functools.partial(
    pl.kernel,
    out_type=jax.ShapeDtypeStruct((_NC, _NP, _D), jnp.float32),
    mesh=_sc_mesh,
    scratch_types=[
        pltpu.VMEM((_CH,), jnp.int32),
        pltpu.VMEM((_CH,), jnp.int32),
        pltpu.VMEM((_CH, _D), jnp.float32),
        pltpu.VMEM((_CH, _D), jnp.float32),
        pltpu.VMEM_SHARED((_NP, _D), jnp.float32),
        pltpu.SemaphoreType.DMA,
    ],
)
def _sc_scatter(w_hbm, s_hbm, src_hbm, dst_hbm, z_hbm, out_hbm,
                src_v, dst_v, w_v, v_v, acc, sem):
    cid = lax.axis_index("c")
    sid = lax.axis_index("s")
    wid = cid * _NS + sid
    pltpu.sync_copy(z_hbm.at[pl.ds(sid * _RPT, _RPT)],
                    acc.at[pl.ds(sid * _RPT, _RPT)])
    plsc.subcore_barrier()

    def chunk(j, carry):
        base = wid * _PERW + j * _CH
        pltpu.sync_copy(src_hbm.at[pl.ds(base, _CH)], src_v)
        pltpu.sync_copy(dst_hbm.at[pl.ds(base, _CH)], dst_v)
        pltpu.sync_copy(w_hbm.at[pl.ds(base, _CH)], w_v)
        pltpu.async_copy(s_hbm.at[src_v], v_v, sem).wait()

        def row(r, c2):
            for c in range(_D // _L):
                sl = pl.ds(c * _L, _L)
                v_v[r, sl] = v_v[r, sl] * w_v[r, sl]
            return c2

        lax.fori_loop(0, _CH, row, 0)
        pltpu.sync_copy(v_v, acc.at[dst_v], add=True)
        return carry

    lax.fori_loop(0, _NCH, chunk, 0)
    plsc.subcore_barrier()
    pltpu.sync_copy(acc.at[pl.ds(sid * _RPT, _RPT)],
                    out_hbm.at[cid, pl.ds(sid * _RPT, _RPT)])


_BN = 1000
_GN = _N // _BN


def _node_body(p_ref, recv_ref, w2T_ref, b2_ref, g_ref, be_ref, w3T_ref,
               b3_ref, o_ref):
    upd = p_ref[0] + p_ref[1]
    y = jnp.dot(upd, w2T_ref[...], preferred_element_type=jnp.float32)
    y = y + b2_ref[...]
    mu = jnp.mean(y, axis=-1, keepdims=True)
    yc = y - mu
    var = jnp.mean(yc * yc, axis=-1, keepdims=True)
    y = yc * lax.rsqrt(var + 1e-5) * g_ref[...] + be_ref[...]
    y = y * jax.nn.sigmoid(y)
    o_ref[...] = (jnp.dot(y, w3T_ref[...], preferred_element_type=jnp.float32)
                  + b3_ref[...] + recv_ref[...])


def _node_mlp(partials, recv, w2T, b2, gamma, beta, w3T, b3):
    return pl.pallas_call(
        _node_body,
        grid=(_GN,),
        in_specs=[
            pl.BlockSpec((_NC, _BN, _D), lambda i: (0, i, 0)),
            pl.BlockSpec((_BN, _D), lambda i: (i, 0)),
            pl.BlockSpec((_D, _D), lambda i: (0, 0)),
            pl.BlockSpec((1, _D), lambda i: (0, 0)),
            pl.BlockSpec((1, _D), lambda i: (0, 0)),
            pl.BlockSpec((1, _D), lambda i: (0, 0)),
            pl.BlockSpec((_D, _D), lambda i: (0, 0)),
            pl.BlockSpec((1, _D), lambda i: (0, 0)),
        ],
        out_specs=pl.BlockSpec((_BN, _D), lambda i: (i, 0)),
        out_shape=jax.ShapeDtypeStruct((_N, _D), jnp.float32),
    )(partials, recv, w2T, b2.reshape(1, _D), gamma.reshape(1, _D),
      beta.reshape(1, _D), w3T, b3.reshape(1, _D))


def kernel(senders, receivers, edge_indices, edge_weights, edge_versors,
           edge_attrs, W1, Wf1, bf1, Wf2, W2, b2, gamma, beta, W3, b3):
    del edge_versors
    s = _lin1(senders[0], W1.T)
    w = _edge_filter(edge_attrs.astype(jnp.bfloat16), edge_weights,
                     Wf1.T.astype(jnp.bfloat16), bf1,
                     Wf2.T.astype(jnp.bfloat16))
    zeros = jnp.zeros((_NP, _D), jnp.float32)
    partials = _sc_scatter(w, s, edge_indices[0], edge_indices[1], zeros)
    return _node_mlp(partials, receivers[0], W2.T, b2, gamma, beta,
                     W3.T, b3)


# speedup vs baseline: 1.4262x; 1.4262x over previous
import functools

import jax
import jax.numpy as jnp
from jax import lax
from jax.experimental import pallas as pl
from jax.experimental.pallas import tpu as pltpu
from jax.experimental.pallas import tpu_sc as plsc

_N = 10000
_E = 320000
_D = 128
_R_CUT = 5.0


def _lin1_body(x_ref, wT_ref, o_ref):
    o_ref[...] = jnp.dot(x_ref[...], wT_ref[...],
                         preferred_element_type=jnp.float32)


def _lin1(x, wT):
    return pl.pallas_call(
        _lin1_body,
        out_shape=jax.ShapeDtypeStruct((_N, _D), jnp.float32),
    )(x, wT)


_BE = 512
_GE = _E // _BE


def _col128(row):
    rid = lax.broadcasted_iota(jnp.int32, (128, 128), 0)
    lid = lax.broadcasted_iota(jnp.int32, (128, 128), 1)
    b = jnp.broadcast_to(row, (128, 128))
    return jnp.sum(jnp.where(rid == lid, b, 0.0), axis=1, keepdims=True)


def _filter_body(attrs_ref, ew_ref, wf1T_ref, bf1_ref, wf2T_ref, w_ref):
    a = attrs_ref[...]
    h = jnp.dot(a, wf1T_ref[...], preferred_element_type=jnp.float32)
    h = h + bf1_ref[...]
    h = h * jax.nn.sigmoid(h)
    w = jnp.dot(h.astype(jnp.bfloat16), wf2T_ref[...],
                preferred_element_type=jnp.float32)
    ew = ew_ref[0]
    c = 0.5 * (jnp.cos(jnp.pi * ew / _R_CUT) + 1.0)
    c = c * (ew < _R_CUT).astype(jnp.float32)
    ccol = jnp.concatenate(
        [_col128(c[r:r + 1, :]) for r in range(_BE // 128)], axis=0)
    w_ref[...] = w * ccol


def _edge_filter(edge_attrs, edge_weights, wf1T, bf1, wf2T):
    ew2 = edge_weights.reshape(_GE, _BE // 128, 128)
    return pl.pallas_call(
        _filter_body,
        grid=(_GE,),
        in_specs=[
            pl.BlockSpec((_BE, _D), lambda i: (i, 0)),
            pl.BlockSpec((1, _BE // 128, 128), lambda i: (i, 0, 0)),
            pl.BlockSpec((_D, _D), lambda i: (0, 0)),
            pl.BlockSpec((1, _D), lambda i: (0, 0)),
            pl.BlockSpec((_D, _D), lambda i: (0, 0)),
        ],
        out_specs=pl.BlockSpec((_BE, _D), lambda i: (i, 0)),
        out_shape=jax.ShapeDtypeStruct((_E, _D), jnp.float32),
    )(edge_attrs, ew2, wf1T, bf1.reshape(1, _D), wf2T)


_NC, _NS, _L = 2, 16, 16
_NW = _NC * _NS
_CH = 64
_PERW = _E // _NW
_NCH = 156
_TL = _PERW - _NCH * _CH
_NP = 10112
_RPT = _NP // _NS

_sc_mesh = plsc.VectorSubcoreMesh(core_axis_name="c", subcore_axis_name="s")


@functools.partial(
    pl.kernel,
    out_type=jax.ShapeDtypeStruct((_NC, _NP, _D), jnp.float32),
    mesh=_sc_mesh,
    scratch_types=[
        pltpu.VMEM((_PERW,), jnp.int32),
        pltpu.VMEM((_CH,), jnp.int32),
        pltpu.VMEM((_CH,), jnp.int32),
        pltpu.VMEM((_CH,), jnp.int32),
        pltpu.VMEM((_CH,), jnp.int32),
        pltpu.VMEM((_TL,), jnp.int32),
        pltpu.VMEM((_TL,), jnp.int32),
        pltpu.VMEM((_CH, _D), jnp.float32),
        pltpu.VMEM((_CH, _D), jnp.float32),
        pltpu.VMEM((_CH, _D), jnp.float32),
        pltpu.VMEM((_CH, _D), jnp.float32),
        pltpu.VMEM_SHARED((_NP, _D), jnp.float32),
        pltpu.SemaphoreType.DMA,
        pltpu.SemaphoreType.DMA,
        pltpu.SemaphoreType.DMA,
        pltpu.SemaphoreType.DMA,
        pltpu.SemaphoreType.DMA,
        pltpu.SemaphoreType.DMA,
    ],
)
def _sc_scatter(w_hbm, s_hbm, pk_hbm, z_hbm, out_hbm,
                pk, srcb0, srcb1, dstb0, dstb1, srct, dstt,
                w0, w1, v0, v1, acc,
                semw0, semw1, semg0, semg1, sems0, sems1):
    cid = lax.axis_index("c")
    sid = lax.axis_index("s")
    wid = cid * _NS + sid
    pltpu.sync_copy(z_hbm, acc.at[pl.ds(sid * _RPT, _RPT)])
    pltpu.sync_copy(pk_hbm.at[wid], pk)
    plsc.subcore_barrier()

    mask = jnp.full((_L,), 0xFFFF, jnp.int32)

    def unpack_idx(j, srcb, dstb):
        for k in range(_CH // _L):
            x = pk[pl.ds(j * _CH + k * _L, _L)]
            srcb[pl.ds(k * _L, _L)] = x & mask
            dstb[pl.ds(k * _L, _L)] = lax.shift_right_logical(x, 16)

    def start_loads(j, w_v, v_v, srcb, dstb, semw, semg):
        unpack_idx(j, srcb, dstb)
        base = wid * _PERW + j * _CH
        pltpu.async_copy(w_hbm.at[pl.ds(base, _CH)], w_v, semw)
        pltpu.async_copy(s_hbm.at[srcb], v_v, semg)

    def wait_loads(j, w_v, v_v, srcb, semw, semg):
        base = wid * _PERW + j * _CH
        pltpu.make_async_copy(w_hbm.at[pl.ds(base, _CH)], w_v, semw).wait()
        pltpu.make_async_copy(s_hbm.at[srcb], v_v, semg).wait()

    def multiply(w_v, v_v, nrow):
        def row(r, c2):
            for c in range(_D // _L):
                sl = pl.ds(c * _L, _L)
                v_v[r, sl] = v_v[r, sl] * w_v[r, sl]
            return c2
        lax.fori_loop(0, nrow, row, 0)

    def start_scatter(v_v, dstb, sems):
        pltpu.async_copy(v_v, acc.at[dstb], sems, add=True)

    def wait_scatter(v_v, dstb, sems):
        pltpu.make_async_copy(v_v, acc.at[dstb], sems).wait()

    start_loads(0, w0, v0, srcb0, dstb0, semw0, semg0)

    def pair(h, carry):
        ja = 2 * h
        jb = ja + 1

        @pl.when(h > 0)
        def _():
            wait_scatter(v1, dstb1, sems1)
        start_loads(jb, w1, v1, srcb1, dstb1, semw1, semg1)
        wait_loads(ja, w0, v0, srcb0, semw0, semg0)
        multiply(w0, v0, _CH)
        start_scatter(v0, dstb0, sems0)

        @pl.when(jb + 1 < _NCH)
        def _():
            wait_scatter(v0, dstb0, sems0)
            start_loads(jb + 1, w0, v0, srcb0, dstb0, semw0, semg0)
        wait_loads(jb, w1, v1, srcb1, semw1, semg1)
        multiply(w1, v1, _CH)
        start_scatter(v1, dstb1, sems1)
        return carry

    lax.fori_loop(0, _NCH // 2, pair, 0)
    wait_scatter(v0, dstb0, sems0)
    wait_scatter(v1, dstb1, sems1)

    xt = pk[pl.ds(_NCH * _CH, _TL)]
    srct[...] = xt & mask
    dstt[...] = lax.shift_right_logical(xt, 16)
    tbase = wid * _PERW + _NCH * _CH
    pltpu.sync_copy(w_hbm.at[pl.ds(tbase, _TL)], w0.at[pl.ds(0, _TL)])
    pltpu.async_copy(s_hbm.at[srct], v0.at[pl.ds(0, _TL)], semg0).wait()
    multiply(w0, v0, _TL)
    pltpu.sync_copy(v0.at[pl.ds(0, _TL)], acc.at[dstt], add=True)

    plsc.subcore_barrier()
    pltpu.sync_copy(acc.at[pl.ds(sid * _RPT, _RPT)],
                    out_hbm.at[cid, pl.ds(sid * _RPT, _RPT)])


_BN = 1000
_GN = _N // _BN


def _node_body(p_ref, recv_ref, w2T_ref, b2_ref, g_ref, be_ref, w3T_ref,
               b3_ref, o_ref):
    upd = p_ref[0] + p_ref[1]
    y = jnp.dot(upd, w2T_ref[...], preferred_element_type=jnp.float32)
    y = y + b2_ref[...]
    mu = jnp.mean(y, axis=-1, keepdims=True)
    yc = y - mu
    var = jnp.mean(yc * yc, axis=-1, keepdims=True)
    y = yc * lax.rsqrt(var + 1e-5) * g_ref[...] + be_ref[...]
    y = y * jax.nn.sigmoid(y)
    o_ref[...] = (jnp.dot(y, w3T_ref[...], preferred_element_type=jnp.float32)
                  + b3_ref[...] + recv_ref[...])


def _node_mlp(partials, recv, w2T, b2, gamma, beta, w3T, b3):
    return pl.pallas_call(
        _node_body,
        grid=(_GN,),
        in_specs=[
            pl.BlockSpec((_NC, _BN, _D), lambda i: (0, i, 0)),
            pl.BlockSpec((_BN, _D), lambda i: (i, 0)),
            pl.BlockSpec((_D, _D), lambda i: (0, 0)),
            pl.BlockSpec((1, _D), lambda i: (0, 0)),
            pl.BlockSpec((1, _D), lambda i: (0, 0)),
            pl.BlockSpec((1, _D), lambda i: (0, 0)),
            pl.BlockSpec((_D, _D), lambda i: (0, 0)),
            pl.BlockSpec((1, _D), lambda i: (0, 0)),
        ],
        out_specs=pl.BlockSpec((_BN, _D), lambda i: (i, 0)),
        out_shape=jax.ShapeDtypeStruct((_N, _D), jnp.float32),
    )(partials, recv, w2T, b2.reshape(1, _D), gamma.reshape(1, _D),
      beta.reshape(1, _D), w3T, b3.reshape(1, _D))


def kernel(senders, receivers, edge_indices, edge_weights, edge_versors,
           edge_attrs, W1, Wf1, bf1, Wf2, W2, b2, gamma, beta, W3, b3):
    del edge_versors
    s = _lin1(senders[0], W1.T)
    w = _edge_filter(edge_attrs.astype(jnp.bfloat16), edge_weights,
                     Wf1.T.astype(jnp.bfloat16), bf1,
                     Wf2.T.astype(jnp.bfloat16))
    zeros = jnp.zeros((_RPT, _D), jnp.float32)
    packed = (edge_indices[0] | (edge_indices[1] << 16)).reshape(_NW, _PERW)
    partials = _sc_scatter(w, s, packed, zeros)
    return _node_mlp(partials, receivers[0], W2.T, b2, gamma, beta,
                     W3.T, b3)


# speedup vs baseline: 2.1662x; 1.5189x over previous
import functools

import jax
import jax.numpy as jnp
import numpy as np
from jax import lax
from jax.experimental import pallas as pl
from jax.experimental.pallas import tpu as pltpu
from jax.experimental.pallas import tpu_sc as plsc

_N = 10000
_E = 320000
_D = 128
_R_CUT = 5.0


def _lin1_body(x_ref, wT_ref, o_ref):
    o_ref[...] = jnp.dot(x_ref[...], wT_ref[...],
                         preferred_element_type=jnp.float32)


def _lin1(x, wT):
    return pl.pallas_call(
        _lin1_body,
        out_shape=jax.ShapeDtypeStruct((_N, _D), jnp.float32),
    )(x, wT)


_BE = 1280
_GE = _E // _BE


def _col128(row):
    rid = lax.broadcasted_iota(jnp.int32, (128, 128), 0)
    lid = lax.broadcasted_iota(jnp.int32, (128, 128), 1)
    b = jnp.broadcast_to(row, (128, 128))
    return jnp.sum(jnp.where(rid == lid, b, 0.0), axis=1, keepdims=True)


def _filter_body(attrs_ref, ew_ref, wf1T_ref, bf1_ref, wf2T_ref, w_ref):
    a = attrs_ref[...].astype(jnp.bfloat16)
    h = jnp.dot(a, wf1T_ref[...], preferred_element_type=jnp.float32)
    h = h + bf1_ref[...]
    h = h * jax.nn.sigmoid(h)
    w = jnp.dot(h.astype(jnp.bfloat16), wf2T_ref[...],
                preferred_element_type=jnp.float32)
    ew = ew_ref[0]
    c = 0.5 * (jnp.cos(jnp.pi * ew / _R_CUT) + 1.0)
    c = c * (ew < _R_CUT).astype(jnp.float32)
    ccol = jnp.concatenate(
        [_col128(c[r:r + 1, :]) for r in range(_BE // 128)], axis=0)
    w_ref[...] = w * ccol


def _edge_filter(edge_attrs, edge_weights, wf1T, bf1, wf2T):
    ew2 = edge_weights.reshape(_GE, _BE // 128, 128)
    return pl.pallas_call(
        _filter_body,
        grid=(_GE,),
        in_specs=[
            pl.BlockSpec((_BE, _D), lambda i: (i, 0)),
            pl.BlockSpec((1, _BE // 128, 128), lambda i: (i, 0, 0)),
            pl.BlockSpec((_D, _D), lambda i: (0, 0)),
            pl.BlockSpec((1, _D), lambda i: (0, 0)),
            pl.BlockSpec((_D, _D), lambda i: (0, 0)),
        ],
        out_specs=pl.BlockSpec((_BE, _D), lambda i: (i, 0)),
        out_shape=jax.ShapeDtypeStruct((_E, _D), jnp.float32),
    )(edge_attrs, ew2, wf1T, bf1.reshape(1, _D), wf2T)


_NC, _NS, _L = 2, 16, 16
_NW = _NC * _NS
_CH = 64
_PERW = _E // _NW
_NCH = 156
_TL = _PERW - _NCH * _CH
_NP = 10112
_RPT = _NP // _NS

_sc_mesh = plsc.VectorSubcoreMesh(core_axis_name="c", subcore_axis_name="s")


@functools.partial(
    pl.kernel,
    out_type=jax.ShapeDtypeStruct((_NC, _NP, _D), jnp.float32),
    mesh=_sc_mesh,
    scratch_types=[
        pltpu.VMEM((_PERW,), jnp.int32),
        pltpu.VMEM((_CH,), jnp.int32),
        pltpu.VMEM((_CH,), jnp.int32),
        pltpu.VMEM((_CH,), jnp.int32),
        pltpu.VMEM((_CH,), jnp.int32),
        pltpu.VMEM((_TL,), jnp.int32),
        pltpu.VMEM((_TL,), jnp.int32),
        pltpu.VMEM((_CH, _D), jnp.float32),
        pltpu.VMEM((_CH, _D), jnp.float32),
        pltpu.VMEM((_CH, _D), jnp.float32),
        pltpu.VMEM((_CH, _D), jnp.float32),
        pltpu.VMEM_SHARED((_NP, _D), jnp.float32),
        pltpu.SemaphoreType.DMA,
        pltpu.SemaphoreType.DMA,
        pltpu.SemaphoreType.DMA,
        pltpu.SemaphoreType.DMA,
        pltpu.SemaphoreType.DMA,
        pltpu.SemaphoreType.DMA,
    ],
)
def _sc_scatter(w_hbm, s_hbm, pk_hbm, z_hbm, out_hbm,
                pk, srcb0, srcb1, dstb0, dstb1, srct, dstt,
                w0, w1, v0, v1, acc,
                semw0, semw1, semg0, semg1, sems0, sems1):
    cid = lax.axis_index("c")
    sid = lax.axis_index("s")
    wid = cid * _NS + sid
    pltpu.sync_copy(z_hbm, acc.at[pl.ds(sid * _RPT, _RPT)])
    pltpu.sync_copy(pk_hbm.at[wid], pk)
    plsc.subcore_barrier()

    mask = jnp.full((_L,), 0xFFFF, jnp.int32)

    def unpack_idx(j, srcb, dstb):
        for k in range(_CH // _L):
            x = pk[pl.ds(j * _CH + k * _L, _L)]
            srcb[pl.ds(k * _L, _L)] = x & mask
            dstb[pl.ds(k * _L, _L)] = lax.shift_right_logical(x, 16)

    def start_loads(j, w_v, v_v, srcb, dstb, semw, semg):
        unpack_idx(j, srcb, dstb)
        base = wid * _PERW + j * _CH
        pltpu.async_copy(w_hbm.at[pl.ds(base, _CH)], w_v, semw)
        pltpu.async_copy(s_hbm.at[srcb], v_v, semg)

    def wait_loads(j, w_v, v_v, srcb, semw, semg):
        base = wid * _PERW + j * _CH
        pltpu.make_async_copy(w_hbm.at[pl.ds(base, _CH)], w_v, semw).wait()
        pltpu.make_async_copy(s_hbm.at[srcb], v_v, semg).wait()

    def multiply(w_v, v_v, nrow):
        def row(r, c2):
            for c in range(_D // _L):
                sl = pl.ds(c * _L, _L)
                v_v[r, sl] = v_v[r, sl] * w_v[r, sl]
            return c2
        lax.fori_loop(0, nrow, row, 0)

    def start_scatter(m_v, dstb, sems):
        pltpu.async_copy(m_v, acc.at[dstb], sems, add=True)

    def wait_scatter(m_v, dstb, sems):
        pltpu.make_async_copy(m_v, acc.at[dstb], sems).wait()

    start_loads(0, w0, v0, srcb0, dstb0, semw0, semg0)

    def pair(h, carry):
        ja = 2 * h
        jb = ja + 1

        @pl.when(h > 0)
        def _():
            wait_scatter(v1, dstb1, sems1)
        start_loads(jb, w1, v1, srcb1, dstb1, semw1, semg1)
        wait_loads(ja, w0, v0, srcb0, semw0, semg0)
        multiply(w0, v0, _CH)
        start_scatter(v0, dstb0, sems0)

        @pl.when(jb + 1 < _NCH)
        def _():
            wait_scatter(v0, dstb0, sems0)
            start_loads(jb + 1, w0, v0, srcb0, dstb0, semw0, semg0)
        wait_loads(jb, w1, v1, srcb1, semw1, semg1)
        multiply(w1, v1, _CH)
        start_scatter(v1, dstb1, sems1)
        return carry

    lax.fori_loop(0, _NCH // 2, pair, 0)
    wait_scatter(v0, dstb0, sems0)
    wait_scatter(v1, dstb1, sems1)

    xt = pk[pl.ds(_NCH * _CH, _TL)]
    srct[...] = xt & mask
    dstt[...] = lax.shift_right_logical(xt, 16)
    tbase = wid * _PERW + _NCH * _CH
    pltpu.sync_copy(w_hbm.at[pl.ds(tbase, _TL)], w0.at[pl.ds(0, _TL)])
    pltpu.async_copy(s_hbm.at[srct], v0.at[pl.ds(0, _TL)], semg0).wait()
    multiply(w0, v0, _TL)
    pltpu.sync_copy(v0.at[pl.ds(0, _TL)], acc.at[dstt], add=True)

    plsc.subcore_barrier()
    pltpu.sync_copy(acc.at[pl.ds(sid * _RPT, _RPT)],
                    out_hbm.at[cid, pl.ds(sid * _RPT, _RPT)])


_BN = 1000
_GN = _N // _BN


def _node_body(p_ref, recv_ref, w2T_ref, b2_ref, g_ref, be_ref, w3T_ref,
               b3_ref, o_ref):
    upd = p_ref[0] + p_ref[1]
    y = jnp.dot(upd, w2T_ref[...], preferred_element_type=jnp.float32)
    y = y + b2_ref[...]
    mu = jnp.mean(y, axis=-1, keepdims=True)
    yc = y - mu
    var = jnp.mean(yc * yc, axis=-1, keepdims=True)
    y = yc * lax.rsqrt(var + 1e-5) * g_ref[...] + be_ref[...]
    y = y * jax.nn.sigmoid(y)
    o_ref[...] = (jnp.dot(y, w3T_ref[...], preferred_element_type=jnp.float32)
                  + b3_ref[...] + recv_ref[...])


def _node_mlp(partials, recv, w2T, b2, gamma, beta, w3T, b3):
    return pl.pallas_call(
        _node_body,
        grid=(_GN,),
        in_specs=[
            pl.BlockSpec((_NC, _BN, _D), lambda i: (0, i, 0)),
            pl.BlockSpec((_BN, _D), lambda i: (i, 0)),
            pl.BlockSpec((_D, _D), lambda i: (0, 0)),
            pl.BlockSpec((1, _D), lambda i: (0, 0)),
            pl.BlockSpec((1, _D), lambda i: (0, 0)),
            pl.BlockSpec((1, _D), lambda i: (0, 0)),
            pl.BlockSpec((_D, _D), lambda i: (0, 0)),
            pl.BlockSpec((1, _D), lambda i: (0, 0)),
        ],
        out_specs=pl.BlockSpec((_BN, _D), lambda i: (i, 0)),
        out_shape=jax.ShapeDtypeStruct((_N, _D), jnp.float32),
    )(partials, recv, w2T, b2.reshape(1, _D), gamma.reshape(1, _D),
      beta.reshape(1, _D), w3T, b3.reshape(1, _D))


def kernel(senders, receivers, edge_indices, edge_weights, edge_versors,
           edge_attrs, W1, Wf1, bf1, Wf2, W2, b2, gamma, beta, W3, b3):
    del edge_versors
    s = _lin1(senders[0], W1.T)
    w = _edge_filter(edge_attrs, edge_weights,
                     Wf1.T.astype(jnp.bfloat16), bf1,
                     Wf2.T.astype(jnp.bfloat16))
    zeros = jnp.zeros((_RPT, _D), jnp.float32)
    packed = (edge_indices[0] | (edge_indices[1] << 16)).reshape(_NW, _PERW)
    partials = _sc_scatter(w, s, packed, zeros)
    return _node_mlp(partials, receivers[0], W2.T, b2, gamma, beta,
                     W3.T, b3)
